# baseline (speedup 1.0000x reference)
import jax
import jax.numpy as jnp
from jax import lax
from jax.experimental import pallas as pl
from jax.experimental.pallas import tpu as pltpu
from jax.experimental.pallas import tpu_sc as plsc

D = 1024
LANES = 16
NC, NS = 2, 16
NW = NC * NS
BATCH, SEQ = 4, 2048
ROWS_PER_W = BATCH * SEQ // NW
W_PER_BATCH = SEQ // ROWS_PER_W
CHUNK = 8
NCHUNK = ROWS_PER_W // CHUNK
NSET = 3
KS = 3
VECS_PER_ROW = D // LANES


def _pe_body(x_hbm, n_hbm, emb_hbm, out_hbm, idx_all,
             rows0, rows1, rows2, xb0, xb1, xb2, spm,
             sg0, sg1, sg2, sx0, sx1, sx2,
             ss0, ss1, ss2, so0, so1, so2):
    rows = [rows0, rows1, rows2]
    xb = [xb0, xb1, xb2]
    sg = [sg0, sg1, sg2]
    sx = [sx0, sx1, sx2]
    ss = [ss0, ss1, ss2]
    so = [so0, so1, so2]

    wid = lax.axis_index("s") * NC + lax.axis_index("c")
    sid = lax.axis_index("s")
    b_i = wid // W_PER_BATCH
    s0 = (wid % W_PER_BATCH) * ROWS_PER_W
    pltpu.sync_copy(n_hbm.at[b_i, pl.ds(s0, ROWS_PER_W)], idx_all)

    gfut = [None] * NSET
    xfut = [None] * NSET
    sfut = [None] * KS
    ofut = [None] * KS

    def issue(t):
        b = t % NSET
        gfut[b] = pltpu.async_copy(
            emb_hbm.at[idx_all.at[pl.ds(t * CHUNK, CHUNK)]], rows[b], sg[b])
        xfut[b] = pltpu.async_copy(
            x_hbm.at[b_i, pl.ds(s0 + t * CHUNK, CHUNK)], xb[b], sx[b])

    def drain_to_hbm(t_prev):
        k = t_prev % KS
        sfut[k].wait()
        sfut[k] = None
        ofut[k] = pltpu.async_copy(
            spm.at[sid, k],
            out_hbm.at[b_i, pl.ds(s0 + t_prev * CHUNK, CHUNK)], so[k])

    issue(0)
    for t in range(NCHUNK):
        b = t % NSET
        if t + 1 < NCHUNK:
            issue(t + 1)
        gfut[b].wait()
        xfut[b].wait()

        def row_body(r, carry):
            @plsc.parallel_loop(0, VECS_PER_ROW, unroll=8)
            def vec_body(j):
                col = j * LANES
                plsc.addupdate(xb[b].at[r, pl.ds(col, LANES)],
                               rows[b][r, pl.ds(col, LANES)])
            return carry
        lax.fori_loop(0, CHUNK, row_body, 0)

        if t > 0:
            drain_to_hbm(t - 1)
        k = t % KS
        if ofut[k] is not None:
            ofut[k].wait()
            ofut[k] = None
        sfut[k] = pltpu.async_copy(xb[b], spm.at[sid, k], ss[k])

    drain_to_hbm(NCHUNK - 1)
    for k in range(KS):
        if ofut[k] is not None:
            ofut[k].wait()


@jax.jit
def kernel(x, n, emb):
    mesh = plsc.VectorSubcoreMesh(
        core_axis_name="c", subcore_axis_name="s",
        num_cores=NC, num_subcores=NS)
    run = pl.kernel(
        _pe_body,
        out_type=jax.ShapeDtypeStruct((BATCH, SEQ, D), jnp.float32),
        mesh=mesh,
        scratch_types=(
            [pltpu.VMEM((ROWS_PER_W,), jnp.int32)]
            + [pltpu.VMEM((CHUNK, D), jnp.float32) for _ in range(2 * NSET)]
            + [pltpu.VMEM_SHARED((NS, KS, CHUNK, D), jnp.float32)]
            + [pltpu.SemaphoreType.DMA for _ in range(2 * NSET + 2 * KS)]
        ),
    )
    return run(x, n.astype(jnp.int32), emb)


# speedup vs baseline: 1.0300x; 1.0300x over previous
import jax
import jax.numpy as jnp
from jax import lax
from jax.experimental import pallas as pl
from jax.experimental.pallas import tpu as pltpu
from jax.experimental.pallas import tpu_sc as plsc

D = 1024
LANES = 16
NC, NS = 2, 16
NW = NC * NS
BATCH, SEQ = 4, 2048
ROWS_PER_W = BATCH * SEQ // NW
W_PER_BATCH = SEQ // ROWS_PER_W
CHUNK = 16
NCHUNK = ROWS_PER_W // CHUNK
NSET = 3
VECS_PER_ROW = D // LANES


def _pe_body(x_hbm, n_hbm, emb_hbm, out_hbm, idx_all,
             rows0, rows1, rows2, xb0, xb1, xb2,
             sg0, sg1, sg2, sx0, sx1, sx2, so0, so1, so2):
    rows = [rows0, rows1, rows2]
    xb = [xb0, xb1, xb2]
    sg = [sg0, sg1, sg2]
    sx = [sx0, sx1, sx2]
    so = [so0, so1, so2]

    wid = lax.axis_index("s") * NC + lax.axis_index("c")
    b_i = wid // W_PER_BATCH
    s0 = (wid % W_PER_BATCH) * ROWS_PER_W
    pltpu.sync_copy(n_hbm.at[b_i, pl.ds(s0, ROWS_PER_W)], idx_all)

    gfut = [None] * NSET
    xfut = [None] * NSET
    ofut = [None] * NSET

    def issue(t):
        b = t % NSET
        r0 = s0 + t * CHUNK
        if ofut[b] is not None:
            ofut[b].wait()
            ofut[b] = None
        gfut[b] = pltpu.async_copy(
            emb_hbm.at[idx_all.at[pl.ds(t * CHUNK, CHUNK)]], rows[b], sg[b])
        xfut[b] = pltpu.async_copy(x_hbm.at[b_i, pl.ds(r0, CHUNK)], xb[b], sx[b])

    issue(0)
    for t in range(NCHUNK):
        b = t % NSET
        if t + 1 < NCHUNK:
            issue(t + 1)
        gfut[b].wait()
        xfut[b].wait()

        def row_body(r, carry):
            @plsc.parallel_loop(0, VECS_PER_ROW, unroll=8)
            def vec_body(j):
                col = j * LANES
                plsc.addupdate(xb[b].at[r, pl.ds(col, LANES)],
                               rows[b][r, pl.ds(col, LANES)])
            return carry
        lax.fori_loop(0, CHUNK, row_body, 0)

        ofut[b] = pltpu.async_copy(
            xb[b], out_hbm.at[b_i, pl.ds(s0 + t * CHUNK, CHUNK)], so[b])

    for b in range(NSET):
        if ofut[b] is not None:
            ofut[b].wait()


@jax.jit
def kernel(x, n, emb):
    mesh = plsc.VectorSubcoreMesh(
        core_axis_name="c", subcore_axis_name="s",
        num_cores=NC, num_subcores=NS)
    run = pl.kernel(
        _pe_body,
        out_type=jax.ShapeDtypeStruct((BATCH, SEQ, D), jnp.float32),
        mesh=mesh,
        scratch_types=(
            [pltpu.VMEM((ROWS_PER_W,), jnp.int32)]
            + [pltpu.VMEM((CHUNK, D), jnp.float32) for _ in range(2 * NSET)]
            + [pltpu.SemaphoreType.DMA for _ in range(3 * NSET)]
        ),
    )
    return run(x, n.astype(jnp.int32), emb)
